# baseline (speedup 1.0000x reference)
import functools

import jax
import jax.numpy as jnp
from jax import lax
from jax.experimental import pallas as pl
from jax.experimental.pallas import tpu as pltpu
from jax.experimental.pallas import tpu_sc as plsc

N_TOKENS = 32768
N_EXPERTS = 64
SHARP = 10.0


_TC_BLOCK = 2048


def _probs_body(pos_ref, ctrs_t_ref, radii_ref, out_ref):
    x = pos_ref[:, 0:1]
    y = pos_ref[:, 1:2]
    z = pos_ref[:, 2:3]
    cx = ctrs_t_ref[0:1, :]
    cy = ctrs_t_ref[1:2, :]
    cz = ctrs_t_ref[2:3, :]
    dx = x - cx
    dy = y - cy
    dz = z - cz
    d2 = dx * dx + dy * dy + dz * dz
    dist = jnp.sqrt(d2 + 1e-12)
    safe_r = jnp.maximum(jnp.abs(radii_ref[0:1, :]), 0.01)
    logits = SHARP * (safe_r - dist)
    m = jnp.max(logits, axis=-1, keepdims=True)
    e = jnp.exp(logits - m)
    s = jnp.sum(e, axis=-1, keepdims=True)
    out_ref[...] = e / s


def _tc_probs(positions, ctrs_t, radii_row):
    grid = (N_TOKENS // _TC_BLOCK,)
    return pl.pallas_call(
        _probs_body,
        grid=grid,
        in_specs=[
            pl.BlockSpec((_TC_BLOCK, 3), lambda i: (i, 0)),
            pl.BlockSpec((3, N_EXPERTS), lambda i: (0, 0)),
            pl.BlockSpec((1, N_EXPERTS), lambda i: (0, 0)),
        ],
        out_specs=pl.BlockSpec((_TC_BLOCK, N_EXPERTS), lambda i: (i, 0)),
        out_shape=jax.ShapeDtypeStruct((N_TOKENS, N_EXPERTS), jnp.float32),
    )(positions, ctrs_t, radii_row)


_NW = 32
_TPW = N_TOKENS // _NW
_LANES = 16
_GROUP = 32


def _ids_body(px_hbm, py_hbm, pz_hbm, cpar_hbm, ids_hbm,
              px_v, py_v, pz_v, ids_v, cpar_v, sem):
    wid = lax.axis_index("s") * 2 + lax.axis_index("c")
    base = wid * _TPW

    pltpu.sync_copy(cpar_hbm, cpar_v)
    pltpu.async_copy(px_hbm.at[pl.ds(base, _TPW)], px_v, sem).wait()
    pltpu.async_copy(py_hbm.at[pl.ds(base, _TPW)], py_v, sem).wait()
    pltpu.async_copy(pz_hbm.at[pl.ds(base, _TPW)], pz_v, sem).wait()

    inf16 = jnp.full((_LANES,), jnp.inf, jnp.float32)
    zero16 = jnp.zeros((_LANES,), jnp.int32)

    @pl.loop(0, _TPW, step=_GROUP)
    def _(t):
        p0x = px_v[pl.ds(t, _LANES)]
        p0y = py_v[pl.ds(t, _LANES)]
        p0z = pz_v[pl.ds(t, _LANES)]
        p1x = px_v[pl.ds(t + _LANES, _LANES)]
        p1y = py_v[pl.ds(t + _LANES, _LANES)]
        p1z = pz_v[pl.ds(t + _LANES, _LANES)]
        best0, bid0 = inf16, zero16
        best1, bid1 = inf16, zero16
        for e in range(N_EXPERTS):
            mx = cpar_v[pl.ds(e * _LANES, _LANES)]
            my = cpar_v[pl.ds((N_EXPERTS + e) * _LANES, _LANES)]
            mz = cpar_v[pl.ds((2 * N_EXPERTS + e) * _LANES, _LANES)]
            cc = cpar_v[pl.ds((3 * N_EXPERTS + e) * _LANES, _LANES)]
            s0 = (p0x * mx + p0y * my) + (p0z * mz + cc)
            s1 = (p1x * mx + p1y * my) + (p1z * mz + cc)
            c0 = s0 < best0
            c1 = s1 < best1
            best0 = jnp.minimum(best0, s0)
            best1 = jnp.minimum(best1, s1)
            bid0 = jnp.where(c0, jnp.int32(e), bid0)
            bid1 = jnp.where(c1, jnp.int32(e), bid1)
        ids_v[pl.ds(t, _LANES)] = bid0
        ids_v[pl.ds(t + _LANES, _LANES)] = bid1

    pltpu.sync_copy(ids_v, ids_hbm.at[pl.ds(base, _TPW)])


@functools.cache
def _get_sc_ids():
    return pl.kernel(
        _ids_body,
        out_type=jax.ShapeDtypeStruct((N_TOKENS,), jnp.int32),
        mesh=plsc.VectorSubcoreMesh(core_axis_name="c", subcore_axis_name="s"),
        scratch_types=[
            pltpu.VMEM((_TPW,), jnp.float32),
            pltpu.VMEM((_TPW,), jnp.float32),
            pltpu.VMEM((_TPW,), jnp.float32),
            pltpu.VMEM((_TPW,), jnp.int32),
            pltpu.VMEM((4 * N_EXPERTS * _LANES,), jnp.float32),
            pltpu.SemaphoreType.DMA,
        ],
    )


def kernel(positions_3d, centers, radii):
    ctrs_t = centers.T
    radii_row = radii.reshape(1, N_EXPERTS)
    cpar = jnp.concatenate(
        [-2.0 * ctrs_t[0], -2.0 * ctrs_t[1], -2.0 * ctrs_t[2],
         jnp.sum(centers * centers, axis=1)], axis=0)
    cpar = jnp.repeat(cpar, _LANES)
    px = positions_3d[:, 0]
    py = positions_3d[:, 1]
    pz = positions_3d[:, 2]

    probs = _tc_probs(positions_3d, ctrs_t, radii_row)
    ids = _get_sc_ids()(px, py, pz, cpar)
    return (probs, ids)
